# baseline (speedup 1.0000x reference)
import functools

import jax
import jax.numpy as jnp
from jax import lax
from jax.experimental import pallas as pl
from jax.experimental.pallas import tpu as pltpu
from jax.experimental.pallas import tpu_sc as plsc

N = 10000
E = 320000
D = 128
H = 64
G = 64
NC = 2
NS = 16
NW = NC * NS
EPT = E // NW
KE = 128
EPTP = 10240
NSTEP = EPTP // KE
NBUF = 4
NPAD = 10240
RPT = NPAD // NS
RZ = 128
NBLK = 10
BN = N // NBLK


def _deg_body(dst_hbm, out_hbm, dst_v, deg_v):
    c = lax.axis_index("c")
    s = lax.axis_index("s")
    wid = c * NS + s
    zeros16 = jnp.zeros((16,), jnp.float32)
    ones16 = jnp.ones((16,), jnp.float32)

    def zero_body(r, _):
        deg_v[pl.ds(r * 16, 16)] = zeros16
        return 0

    lax.fori_loop(0, NPAD // 16, zero_body, 0)
    pltpu.sync_copy(dst_hbm.at[wid], dst_v)

    def step(j, _):
        for i in range(KE // 16):
            idx = dst_v[j, pl.ds(i * 16, 16)]
            plsc.addupdate_scatter(deg_v, [idx], ones16)
        return 0

    lax.fori_loop(0, NSTEP, step, 0)
    pltpu.sync_copy(deg_v, out_hbm.at[pl.ds(wid * NPAD, NPAD)])


def _edge_body(q_hbm, src_hbm, dst_hbm, out_hbm,
               src_v, dst_v, rows, gsems, ssems, zeros_v, acc_sh):
    c = lax.axis_index("c")
    s = lax.axis_index("s")
    wid = c * NS + s
    zeros16 = jnp.zeros((16,), jnp.float32)

    def zero_body(r, _):
        for i in range(H // 16):
            zeros_v[r, pl.ds(i * 16, 16)] = zeros16
        return 0

    lax.fori_loop(0, RZ, zero_body, 0)
    for t in range(RPT // RZ):
        pltpu.sync_copy(zeros_v, acc_sh.at[pl.ds(s * RPT + t * RZ, RZ)])
    plsc.subcore_barrier()

    pltpu.sync_copy(src_hbm.at[wid], src_v)
    pltpu.sync_copy(dst_hbm.at[wid], dst_v)

    def fire_gather(b, j):
        pltpu.async_copy(q_hbm.at[src_v.at[j]], rows.at[b], gsems[b])

    def wait_gather(b):
        pltpu.make_async_copy(
            q_hbm.at[pl.ds(0, KE)], rows.at[b], gsems[b]).wait()

    fire_gather(0, 0)

    def pair(j2, _):
        j = 2 * j2
        fire_gather(1, j + 1)
        wait_gather(0)
        pltpu.sync_copy(rows.at[0], acc_sh.at[dst_v.at[j]], add=True)
        fire_gather(0, j + 2)
        wait_gather(1)
        pltpu.sync_copy(rows.at[1], acc_sh.at[dst_v.at[j + 1]], add=True)
        return 0

    lax.fori_loop(0, NSTEP // 2 - 1, pair, 0)
    fire_gather(1, NSTEP - 1)
    wait_gather(0)
    pltpu.sync_copy(rows.at[0], acc_sh.at[dst_v.at[NSTEP - 2]], add=True)
    wait_gather(1)
    pltpu.sync_copy(rows.at[1], acc_sh.at[dst_v.at[NSTEP - 1]], add=True)

    plsc.subcore_barrier()
    for t in range(RPT // RZ):
        sl = pl.ds(s * RPT + t * RZ, RZ)
        pltpu.sync_copy(acc_sh.at[sl], out_hbm.at[c, sl])


@functools.lru_cache(maxsize=None)
def _sc_kernels():
    mesh = plsc.VectorSubcoreMesh(
        core_axis_name="c", subcore_axis_name="s",
        num_cores=NC, num_subcores=NS)
    params = pltpu.CompilerParams(needs_layout_passes=False,
                                  use_tc_tiling_on_sc=False)
    deg = pl.kernel(
        _deg_body,
        out_type=jax.ShapeDtypeStruct((NW * NPAD,), jnp.float32),
        mesh=mesh,
        compiler_params=params,
        scratch_types=[
            pltpu.VMEM((NSTEP, KE), jnp.int32),
            pltpu.VMEM((NPAD,), jnp.float32),
        ],
    )
    edge = pl.kernel(
        _edge_body,
        out_type=jax.ShapeDtypeStruct((NC, NPAD, H), jnp.float32),
        mesh=mesh,
        compiler_params=params,
        scratch_types=[
            pltpu.VMEM((NSTEP, KE), jnp.int32),
            pltpu.VMEM((NSTEP, KE), jnp.int32),
            pltpu.VMEM((2, KE, H), jnp.float32),
            [pltpu.SemaphoreType.DMA] * 2,
            [pltpu.SemaphoreType.DMA] * 2,
            pltpu.VMEM((RZ, H), jnp.float32),
            pltpu.VMEM_SHARED((NPAD, H), jnp.float32),
        ],
    )
    return deg, edge


def _prescale_body(x_ref, w_ref, degt_ref, q_ref, dinv_ref):
    deg = jnp.sum(degt_ref[...], axis=1, keepdims=True) + 1.0
    dinv = lax.rsqrt(deg)
    q_ref[...] = jnp.dot(x_ref[...], w_ref[...],
                         preferred_element_type=jnp.float32) * dinv
    dinv_ref[...] = dinv


_prescale = pl.pallas_call(
    _prescale_body,
    grid=(NBLK,),
    in_specs=[
        pl.BlockSpec((BN, D), lambda i: (i, 0)),
        pl.BlockSpec((D, H), lambda i: (0, 0)),
        pl.BlockSpec((BN, NW), lambda i: (i, 0)),
    ],
    out_specs=[
        pl.BlockSpec((BN, H), lambda i: (i, 0)),
        pl.BlockSpec((BN, 1), lambda i: (i, 0)),
    ],
    out_shape=[
        jax.ShapeDtypeStruct((N, H), jnp.float32),
        jax.ShapeDtypeStruct((N, 1), jnp.float32),
    ],
)


def _mid_body(acc_ref, q1_ref, dinv_ref, b1_ref, w2_ref, q2_ref):
    dinv = dinv_ref[...]
    a = acc_ref[0] + acc_ref[1] + q1_ref[...]
    h1 = jnp.maximum(a * dinv + b1_ref[...], 0.0)
    q2_ref[...] = jnp.dot(h1, w2_ref[...],
                          preferred_element_type=jnp.float32) * dinv


_mid = pl.pallas_call(
    _mid_body,
    grid=(NBLK,),
    in_specs=[
        pl.BlockSpec((NC, BN, H), lambda i: (0, i, 0)),
        pl.BlockSpec((BN, H), lambda i: (i, 0)),
        pl.BlockSpec((BN, 1), lambda i: (i, 0)),
        pl.BlockSpec((1, H), lambda i: (0, 0)),
        pl.BlockSpec((H, H), lambda i: (0, 0)),
    ],
    out_specs=pl.BlockSpec((BN, H), lambda i: (i, 0)),
    out_shape=jax.ShapeDtypeStruct((N, H), jnp.float32),
)


def _final_body(acc_ref, q2_ref, dinv_ref, b2_ref, batch_ref,
                wc1_ref, bc1_ref, wc2_ref, bc2_ref, out_ref,
                sums_scr, cnt_scr):
    i = pl.program_id(0)

    @pl.when(i == 0)
    def _():
        sums_scr[...] = jnp.zeros((G, H), jnp.float32)
        cnt_scr[...] = jnp.zeros((G, H), jnp.float32)

    a = acc_ref[0] + acc_ref[1] + q2_ref[...]
    h2 = jnp.maximum(a * dinv_ref[...] + b2_ref[...], 0.0)
    gids = lax.broadcasted_iota(jnp.int32, (1, G), 1)
    onehot = (batch_ref[...] == gids).astype(jnp.float32)
    dn = (((0,), (0,)), ((), ()))
    sums_scr[...] += lax.dot_general(onehot, h2, dn,
                                     preferred_element_type=jnp.float32)
    cnt_scr[...] += lax.dot_general(onehot, jnp.ones_like(h2), dn,
                                    preferred_element_type=jnp.float32)

    @pl.when(i == NBLK - 1)
    def _():
        pooled = sums_scr[...] / jnp.maximum(cnt_scr[...], 1.0)
        z = jnp.maximum(jnp.dot(pooled, wc1_ref[...],
                                preferred_element_type=jnp.float32)
                        + bc1_ref[...], 0.0)
        logit = jnp.dot(z, wc2_ref[...],
                        preferred_element_type=jnp.float32) + bc2_ref[...]
        out_ref[...] = 1.0 / (1.0 + jnp.exp(-logit))


_final = pl.pallas_call(
    _final_body,
    grid=(NBLK,),
    in_specs=[
        pl.BlockSpec((NC, BN, H), lambda i: (0, i, 0)),
        pl.BlockSpec((BN, H), lambda i: (i, 0)),
        pl.BlockSpec((BN, 1), lambda i: (i, 0)),
        pl.BlockSpec((1, H), lambda i: (0, 0)),
        pl.BlockSpec((BN, 1), lambda i: (i, 0)),
        pl.BlockSpec((H, H), lambda i: (0, 0)),
        pl.BlockSpec((1, H), lambda i: (0, 0)),
        pl.BlockSpec((H, 1), lambda i: (0, 0)),
        pl.BlockSpec((1, 1), lambda i: (0, 0)),
    ],
    out_specs=pl.BlockSpec((G, 1), lambda i: (0, 0)),
    out_shape=jax.ShapeDtypeStruct((G, 1), jnp.float32),
    scratch_shapes=[
        pltpu.VMEM((G, H), jnp.float32),
        pltpu.VMEM((G, H), jnp.float32),
    ],
    compiler_params=pltpu.CompilerParams(
        dimension_semantics=("arbitrary",)),
)


def kernel(x, edge_index, batch, W1, b1, W2, b2, Wc1, bc1, Wc2, bc2):
    deg_k, edge_k = _sc_kernels()
    pad_src = jnp.zeros((NW, EPTP - EPT), jnp.int32)
    pad_dst = jnp.full((NW, EPTP - EPT), N, jnp.int32)
    src3 = jnp.concatenate(
        [edge_index[0].reshape(NW, EPT), pad_src], axis=1).reshape(
            NW, NSTEP, KE)
    dst3 = jnp.concatenate(
        [edge_index[1].reshape(NW, EPT), pad_dst], axis=1).reshape(
            NW, NSTEP, KE)
    degp = deg_k(dst3)
    degt = degp.reshape(NW, NPAD).T
    q1, dinv = _prescale(x, W1, degt)
    acc1 = edge_k(q1, src3, dst3)
    q2 = _mid(acc1, q1, dinv, b1.reshape(1, H), W2)
    acc2 = edge_k(q2, src3, dst3)
    out = _final(acc2, q2, dinv, b2.reshape(1, H), batch.reshape(N, 1),
                 Wc1, bc1.reshape(1, H), Wc2, bc2.reshape(1, 1))
    return out


# speedup vs baseline: 1.0001x; 1.0001x over previous
import functools

import jax
import jax.numpy as jnp
from jax import lax
from jax.experimental import pallas as pl
from jax.experimental.pallas import tpu as pltpu
from jax.experimental.pallas import tpu_sc as plsc

N = 10000
E = 320000
D = 128
H = 64
G = 64
NC = 2
NS = 16
NW = NC * NS
EPT = E // NW
KE = 128
EPTP = 10240
NSTEP = EPTP // KE
NBUF = 4
NPAD = 10240
RPT = NPAD // NS
RZ = 128
NBLK = 10
BN = N // NBLK


def _deg_body(dst_hbm, out_hbm, dst_v, deg_v):
    c = lax.axis_index("c")
    s = lax.axis_index("s")
    wid = c * NS + s
    zeros16 = jnp.zeros((16,), jnp.float32)
    ones16 = jnp.ones((16,), jnp.float32)

    def zero_body(r, _):
        deg_v[pl.ds(r * 16, 16)] = zeros16
        return 0

    lax.fori_loop(0, NPAD // 16, zero_body, 0)
    pltpu.sync_copy(dst_hbm.at[wid], dst_v)

    def step(j, _):
        for i in range(KE // 16):
            idx = dst_v[j, pl.ds(i * 16, 16)]
            plsc.addupdate_scatter(deg_v, [idx], ones16)
        return 0

    lax.fori_loop(0, NSTEP, step, 0)
    pltpu.sync_copy(deg_v, out_hbm.at[pl.ds(wid * NPAD, NPAD)])


def _edge_body(q_hbm, src_hbm, dst_hbm, out_hbm,
               src_v, dst_v, rows, gsems, ssems, zeros_v, acc_sh):
    c = lax.axis_index("c")
    s = lax.axis_index("s")
    wid = c * NS + s
    zeros16 = jnp.zeros((16,), jnp.float32)

    def zero_body(r, _):
        for i in range(H // 16):
            zeros_v[r, pl.ds(i * 16, 16)] = zeros16
        return 0

    lax.fori_loop(0, RZ, zero_body, 0)
    for t in range(RPT // RZ):
        pltpu.sync_copy(zeros_v, acc_sh.at[pl.ds(s * RPT + t * RZ, RZ)])
    plsc.subcore_barrier()

    pltpu.sync_copy(src_hbm.at[wid], src_v)
    pltpu.sync_copy(dst_hbm.at[wid], dst_v)

    def fire_gather(b, j):
        pltpu.async_copy(q_hbm.at[src_v.at[j]], rows.at[b], gsems[b])

    def wait_gather(b):
        pltpu.make_async_copy(
            q_hbm.at[pl.ds(0, KE)], rows.at[b], gsems[b]).wait()

    fire_gather(0, 0)

    def pair(j2, _):
        j = 2 * j2
        fire_gather(1, j + 1)
        wait_gather(0)
        pltpu.sync_copy(rows.at[0], acc_sh.at[dst_v.at[j]], add=True)
        fire_gather(0, j + 2)
        wait_gather(1)
        pltpu.sync_copy(rows.at[1], acc_sh.at[dst_v.at[j + 1]], add=True)
        return 0

    lax.fori_loop(0, NSTEP // 2 - 1, pair, 0)
    fire_gather(1, NSTEP - 1)
    wait_gather(0)
    pltpu.sync_copy(rows.at[0], acc_sh.at[dst_v.at[NSTEP - 2]], add=True)
    wait_gather(1)
    pltpu.sync_copy(rows.at[1], acc_sh.at[dst_v.at[NSTEP - 1]], add=True)

    plsc.subcore_barrier()
    for t in range(RPT // RZ):
        sl = pl.ds(s * RPT + t * RZ, RZ)
        pltpu.sync_copy(acc_sh.at[sl], out_hbm.at[c, sl])


@functools.lru_cache(maxsize=None)
def _sc_kernels():
    mesh = plsc.VectorSubcoreMesh(
        core_axis_name="c", subcore_axis_name="s",
        num_cores=NC, num_subcores=NS)
    params = pltpu.CompilerParams(needs_layout_passes=False,
                                  use_tc_tiling_on_sc=False)
    deg = pl.kernel(
        _deg_body,
        out_type=jax.ShapeDtypeStruct((NW * NPAD,), jnp.float32),
        mesh=mesh,
        compiler_params=params,
        scratch_types=[
            pltpu.VMEM((NSTEP, KE), jnp.int32),
            pltpu.VMEM((NPAD,), jnp.float32),
        ],
    )
    edge = pl.kernel(
        _edge_body,
        out_type=jax.ShapeDtypeStruct((NC, NPAD, H), jnp.float32),
        mesh=mesh,
        compiler_params=params,
        scratch_types=[
            pltpu.VMEM((NSTEP, KE), jnp.int32),
            pltpu.VMEM((NSTEP, KE), jnp.int32),
            pltpu.VMEM((2, KE, H), jnp.float32),
            [pltpu.SemaphoreType.DMA] * 2,
            [pltpu.SemaphoreType.DMA] * 2,
            pltpu.VMEM((RZ, H), jnp.float32),
            pltpu.VMEM_SHARED((NPAD, H), jnp.float32),
        ],
    )
    return deg, edge


def _prescale_body(x_ref, w_ref, degt_ref, q_ref, dinv_ref):
    deg = jnp.sum(degt_ref[...], axis=1, keepdims=True) + 1.0
    dinv = lax.rsqrt(deg)
    q_ref[...] = jnp.dot(x_ref[...], w_ref[...],
                         preferred_element_type=jnp.float32) * dinv
    dinv_ref[...] = dinv


_prescale = pl.pallas_call(
    _prescale_body,
    grid=(NBLK,),
    in_specs=[
        pl.BlockSpec((BN, D), lambda i: (i, 0)),
        pl.BlockSpec((D, H), lambda i: (0, 0)),
        pl.BlockSpec((BN, NW), lambda i: (i, 0)),
    ],
    out_specs=[
        pl.BlockSpec((BN, H), lambda i: (i, 0)),
        pl.BlockSpec((BN, 1), lambda i: (i, 0)),
    ],
    out_shape=[
        jax.ShapeDtypeStruct((N, H), jnp.float32),
        jax.ShapeDtypeStruct((N, 1), jnp.float32),
    ],
)


def _mid_body(acc_ref, q1_ref, dinv_ref, b1_ref, w2_ref, q2_ref):
    dinv = dinv_ref[...]
    a = acc_ref[0] + acc_ref[1] + q1_ref[...]
    h1 = jnp.maximum(a * dinv + b1_ref[...], 0.0)
    q2_ref[...] = jnp.dot(h1, w2_ref[...],
                          preferred_element_type=jnp.float32) * dinv


_mid = pl.pallas_call(
    _mid_body,
    grid=(NBLK,),
    in_specs=[
        pl.BlockSpec((NC, BN, H), lambda i: (0, i, 0)),
        pl.BlockSpec((BN, H), lambda i: (i, 0)),
        pl.BlockSpec((BN, 1), lambda i: (i, 0)),
        pl.BlockSpec((1, H), lambda i: (0, 0)),
        pl.BlockSpec((H, H), lambda i: (0, 0)),
    ],
    out_specs=pl.BlockSpec((BN, H), lambda i: (i, 0)),
    out_shape=jax.ShapeDtypeStruct((N, H), jnp.float32),
)


def _final_body(acc_ref, q2_ref, dinv_ref, b2_ref, batch_ref,
                wc1_ref, bc1_ref, wc2_ref, bc2_ref, out_ref,
                sums_scr, cnt_scr):
    i = pl.program_id(0)

    @pl.when(i == 0)
    def _():
        sums_scr[...] = jnp.zeros((G, H), jnp.float32)
        cnt_scr[...] = jnp.zeros((G, H), jnp.float32)

    a = acc_ref[0] + acc_ref[1] + q2_ref[...]
    h2 = jnp.maximum(a * dinv_ref[...] + b2_ref[...], 0.0)
    gids = lax.broadcasted_iota(jnp.int32, (1, G), 1)
    onehot = (batch_ref[...] == gids).astype(jnp.float32)
    dn = (((0,), (0,)), ((), ()))
    sums_scr[...] += lax.dot_general(onehot, h2, dn,
                                     preferred_element_type=jnp.float32)
    cnt_scr[...] += lax.dot_general(onehot, jnp.ones_like(h2), dn,
                                    preferred_element_type=jnp.float32)

    @pl.when(i == NBLK - 1)
    def _():
        pooled = sums_scr[...] / jnp.maximum(cnt_scr[...], 1.0)
        z = jnp.maximum(jnp.dot(pooled, wc1_ref[...],
                                preferred_element_type=jnp.float32)
                        + bc1_ref[...], 0.0)
        logit = jnp.dot(z, wc2_ref[...],
                        preferred_element_type=jnp.float32) + bc2_ref[...]
        out_ref[...] = 1.0 / (1.0 + jnp.exp(-logit))


_final = pl.pallas_call(
    _final_body,
    grid=(NBLK,),
    in_specs=[
        pl.BlockSpec((NC, BN, H), lambda i: (0, i, 0)),
        pl.BlockSpec((BN, H), lambda i: (i, 0)),
        pl.BlockSpec((BN, 1), lambda i: (i, 0)),
        pl.BlockSpec((1, H), lambda i: (0, 0)),
        pl.BlockSpec((BN, 1), lambda i: (i, 0)),
        pl.BlockSpec((H, H), lambda i: (0, 0)),
        pl.BlockSpec((1, H), lambda i: (0, 0)),
        pl.BlockSpec((H, 1), lambda i: (0, 0)),
        pl.BlockSpec((1, 1), lambda i: (0, 0)),
    ],
    out_specs=pl.BlockSpec((G, 1), lambda i: (0, 0)),
    out_shape=jax.ShapeDtypeStruct((G, 1), jnp.float32),
    scratch_shapes=[
        pltpu.VMEM((G, H), jnp.float32),
        pltpu.VMEM((G, H), jnp.float32),
    ],
    compiler_params=pltpu.CompilerParams(
        dimension_semantics=("arbitrary",)),
)


def kernel(x, edge_index, batch, W1, b1, W2, b2, Wc1, bc1, Wc2, bc2):
    deg_k, edge_k = _sc_kernels()
    pad_src = jnp.zeros((NW, EPTP - EPT), jnp.int32)
    pad_dst = jnp.broadcast_to(
        N + jnp.arange(EPTP - EPT, dtype=jnp.int32), (NW, EPTP - EPT))
    src3 = jnp.concatenate(
        [edge_index[0].reshape(NW, EPT), pad_src], axis=1).reshape(
            NW, NSTEP, KE)
    dst3 = jnp.concatenate(
        [edge_index[1].reshape(NW, EPT), pad_dst], axis=1).reshape(
            NW, NSTEP, KE)
    degp = deg_k(dst3)
    degt = degp.reshape(NW, NPAD).T
    q1, dinv = _prescale(x, W1, degt)
    acc1 = edge_k(q1, src3, dst3)
    q2 = _mid(acc1, q1, dinv, b1.reshape(1, H), W2)
    acc2 = edge_k(q2, src3, dst3)
    out = _final(acc2, q2, dinv, b2.reshape(1, H), batch.reshape(N, 1),
                 Wc1, bc1.reshape(1, H), Wc2, bc2.reshape(1, 1))
    return out


# speedup vs baseline: 1.9455x; 1.9453x over previous
import functools

import jax
import jax.numpy as jnp
from jax import lax
from jax.experimental import pallas as pl
from jax.experimental.pallas import tpu as pltpu
from jax.experimental.pallas import tpu_sc as plsc

N = 10000
E = 320000
D = 128
H = 64
G = 64
NC = 2
NS = 16
NW = NC * NS
EPT = E // NW
KE = 80
NSTEP = EPT // KE
NPAD = 10240
RPT = NPAD // NS
RZ = 128
NBLK = 10
BN = N // NBLK


def _deg_body(dst_hbm, out_hbm, dst_v, deg_v):
    c = lax.axis_index("c")
    s = lax.axis_index("s")
    wid = c * NS + s
    zeros16 = jnp.zeros((16,), jnp.float32)
    ones16 = jnp.ones((16,), jnp.float32)

    def zero_body(r, _):
        deg_v[pl.ds(r * 16, 16)] = zeros16
        return 0

    lax.fori_loop(0, NPAD // 16, zero_body, 0)
    pltpu.sync_copy(dst_hbm.at[wid], dst_v)

    def step(j, _):
        for i in range(KE // 16):
            idx = dst_v[j, pl.ds(i * 16, 16)]
            plsc.addupdate_scatter(deg_v, [idx], ones16)
        return 0

    lax.fori_loop(0, NSTEP, step, 0)
    pltpu.sync_copy(deg_v, out_hbm.at[pl.ds(wid * NPAD, NPAD)])


def _edge_body(q_hbm, src_hbm, dst_hbm, out_hbm,
               src_v, dst_v, rows, gsems, ssems, zeros_v, acc_sh):
    c = lax.axis_index("c")
    s = lax.axis_index("s")
    wid = c * NS + s
    zeros16 = jnp.zeros((16,), jnp.float32)

    def zero_body(r, _):
        for i in range(H // 16):
            zeros_v[r, pl.ds(i * 16, 16)] = zeros16
        return 0

    lax.fori_loop(0, RZ, zero_body, 0)
    for t in range(RPT // RZ):
        pltpu.sync_copy(zeros_v, acc_sh.at[pl.ds(s * RPT + t * RZ, RZ)])
    plsc.subcore_barrier()

    pltpu.sync_copy(src_hbm.at[wid], src_v)
    pltpu.sync_copy(dst_hbm.at[wid], dst_v)

    def fire_gather(b, j):
        pltpu.async_copy(q_hbm.at[src_v.at[j]], rows.at[b], gsems[b])

    def wait_gather(b):
        pltpu.make_async_copy(
            q_hbm.at[pl.ds(0, KE)], rows.at[b], gsems[b]).wait()

    fire_gather(0, 0)

    def pair(j2, _):
        j = 2 * j2
        fire_gather(1, j + 1)
        wait_gather(0)
        pltpu.sync_copy(rows.at[0], acc_sh.at[dst_v.at[j]], add=True)
        fire_gather(0, j + 2)
        wait_gather(1)
        pltpu.sync_copy(rows.at[1], acc_sh.at[dst_v.at[j + 1]], add=True)
        return 0

    lax.fori_loop(0, (NSTEP - 1) // 2, pair, 0)
    wait_gather(0)
    pltpu.sync_copy(rows.at[0], acc_sh.at[dst_v.at[NSTEP - 1]], add=True)

    plsc.subcore_barrier()
    for t in range(RPT // RZ):
        sl = pl.ds(s * RPT + t * RZ, RZ)
        pltpu.sync_copy(acc_sh.at[sl], out_hbm.at[c, sl])


@functools.lru_cache(maxsize=None)
def _sc_kernels():
    mesh = plsc.VectorSubcoreMesh(
        core_axis_name="c", subcore_axis_name="s",
        num_cores=NC, num_subcores=NS)
    params = pltpu.CompilerParams(needs_layout_passes=False,
                                  use_tc_tiling_on_sc=False)
    deg = pl.kernel(
        _deg_body,
        out_type=jax.ShapeDtypeStruct((NW * NPAD,), jnp.float32),
        mesh=mesh,
        compiler_params=params,
        scratch_types=[
            pltpu.VMEM((NSTEP, KE), jnp.int32),
            pltpu.VMEM((NPAD,), jnp.float32),
        ],
    )
    edge = pl.kernel(
        _edge_body,
        out_type=jax.ShapeDtypeStruct((NC, NPAD, H), jnp.float32),
        mesh=mesh,
        compiler_params=params,
        scratch_types=[
            pltpu.VMEM((NSTEP, KE), jnp.int32),
            pltpu.VMEM((NSTEP, KE), jnp.int32),
            pltpu.VMEM((2, KE, H), jnp.float32),
            [pltpu.SemaphoreType.DMA] * 2,
            [pltpu.SemaphoreType.DMA] * 2,
            pltpu.VMEM((RZ, H), jnp.float32),
            pltpu.VMEM_SHARED((NPAD, H), jnp.float32),
        ],
    )
    return deg, edge


def _prescale_body(x_ref, w_ref, degt_ref, q_ref, dinv_ref):
    deg = jnp.sum(degt_ref[...], axis=1, keepdims=True) + 1.0
    dinv = lax.rsqrt(deg)
    q_ref[...] = jnp.dot(x_ref[...], w_ref[...],
                         preferred_element_type=jnp.float32) * dinv
    dinv_ref[...] = dinv


_prescale = pl.pallas_call(
    _prescale_body,
    grid=(NBLK,),
    in_specs=[
        pl.BlockSpec((BN, D), lambda i: (i, 0)),
        pl.BlockSpec((D, H), lambda i: (0, 0)),
        pl.BlockSpec((BN, NW), lambda i: (i, 0)),
    ],
    out_specs=[
        pl.BlockSpec((BN, H), lambda i: (i, 0)),
        pl.BlockSpec((BN, 1), lambda i: (i, 0)),
    ],
    out_shape=[
        jax.ShapeDtypeStruct((N, H), jnp.float32),
        jax.ShapeDtypeStruct((N, 1), jnp.float32),
    ],
)


def _mid_body(acc_ref, q1_ref, dinv_ref, b1_ref, w2_ref, q2_ref):
    dinv = dinv_ref[...]
    a = acc_ref[0] + acc_ref[1] + q1_ref[...]
    h1 = jnp.maximum(a * dinv + b1_ref[...], 0.0)
    q2_ref[...] = jnp.dot(h1, w2_ref[...],
                          preferred_element_type=jnp.float32) * dinv


_mid = pl.pallas_call(
    _mid_body,
    grid=(NBLK,),
    in_specs=[
        pl.BlockSpec((NC, BN, H), lambda i: (0, i, 0)),
        pl.BlockSpec((BN, H), lambda i: (i, 0)),
        pl.BlockSpec((BN, 1), lambda i: (i, 0)),
        pl.BlockSpec((1, H), lambda i: (0, 0)),
        pl.BlockSpec((H, H), lambda i: (0, 0)),
    ],
    out_specs=pl.BlockSpec((BN, H), lambda i: (i, 0)),
    out_shape=jax.ShapeDtypeStruct((N, H), jnp.float32),
)


def _final_body(acc_ref, q2_ref, dinv_ref, b2_ref, batch_ref,
                wc1_ref, bc1_ref, wc2_ref, bc2_ref, out_ref,
                sums_scr, cnt_scr):
    i = pl.program_id(0)

    @pl.when(i == 0)
    def _():
        sums_scr[...] = jnp.zeros((G, H), jnp.float32)
        cnt_scr[...] = jnp.zeros((G, H), jnp.float32)

    a = acc_ref[0] + acc_ref[1] + q2_ref[...]
    h2 = jnp.maximum(a * dinv_ref[...] + b2_ref[...], 0.0)
    gids = lax.broadcasted_iota(jnp.int32, (1, G), 1)
    onehot = (batch_ref[...] == gids).astype(jnp.float32)
    dn = (((0,), (0,)), ((), ()))
    sums_scr[...] += lax.dot_general(onehot, h2, dn,
                                     preferred_element_type=jnp.float32)
    cnt_scr[...] += lax.dot_general(onehot, jnp.ones_like(h2), dn,
                                    preferred_element_type=jnp.float32)

    @pl.when(i == NBLK - 1)
    def _():
        pooled = sums_scr[...] / jnp.maximum(cnt_scr[...], 1.0)
        z = jnp.maximum(jnp.dot(pooled, wc1_ref[...],
                                preferred_element_type=jnp.float32)
                        + bc1_ref[...], 0.0)
        logit = jnp.dot(z, wc2_ref[...],
                        preferred_element_type=jnp.float32) + bc2_ref[...]
        out_ref[...] = 1.0 / (1.0 + jnp.exp(-logit))


_final = pl.pallas_call(
    _final_body,
    grid=(NBLK,),
    in_specs=[
        pl.BlockSpec((NC, BN, H), lambda i: (0, i, 0)),
        pl.BlockSpec((BN, H), lambda i: (i, 0)),
        pl.BlockSpec((BN, 1), lambda i: (i, 0)),
        pl.BlockSpec((1, H), lambda i: (0, 0)),
        pl.BlockSpec((BN, 1), lambda i: (i, 0)),
        pl.BlockSpec((H, H), lambda i: (0, 0)),
        pl.BlockSpec((1, H), lambda i: (0, 0)),
        pl.BlockSpec((H, 1), lambda i: (0, 0)),
        pl.BlockSpec((1, 1), lambda i: (0, 0)),
    ],
    out_specs=pl.BlockSpec((G, 1), lambda i: (0, 0)),
    out_shape=jax.ShapeDtypeStruct((G, 1), jnp.float32),
    scratch_shapes=[
        pltpu.VMEM((G, H), jnp.float32),
        pltpu.VMEM((G, H), jnp.float32),
    ],
    compiler_params=pltpu.CompilerParams(
        dimension_semantics=("arbitrary",)),
)


def kernel(x, edge_index, batch, W1, b1, W2, b2, Wc1, bc1, Wc2, bc2):
    deg_k, edge_k = _sc_kernels()
    src3 = edge_index[0].reshape(NW, NSTEP, KE)
    dst3 = edge_index[1].reshape(NW, NSTEP, KE)
    degp = deg_k(dst3)
    degt = degp.reshape(NW, NPAD).T
    q1, dinv = _prescale(x, W1, degt)
    acc1 = edge_k(q1, src3, dst3)
    q2 = _mid(acc1, q1, dinv, b1.reshape(1, H), W2)
    acc2 = edge_k(q2, src3, dst3)
    out = _final(acc2, q2, dinv, b2.reshape(1, H), batch.reshape(N, 1),
                 Wc1, bc1.reshape(1, H), Wc2, bc2.reshape(1, 1))
    return out


# speedup vs baseline: 1.9758x; 1.0156x over previous
import functools

import jax
import jax.numpy as jnp
from jax import lax
from jax.experimental import pallas as pl
from jax.experimental.pallas import tpu as pltpu
from jax.experimental.pallas import tpu_sc as plsc

N = 10000
E = 320000
D = 128
H = 64
G = 64
NC = 2
NS = 16
NW = NC * NS
EPT = E // NW
KE = 80
NSTEP = EPT // KE
NPAD = 10240
RPT = NPAD // NS
RZ = 128
NBLK = 10
BN = N // NBLK


def _deg_body(dst_hbm, out_hbm, dst_v, deg_v):
    c = lax.axis_index("c")
    s = lax.axis_index("s")
    wid = c * NS + s
    zeros16 = jnp.zeros((16,), jnp.float32)
    ones16 = jnp.ones((16,), jnp.float32)

    def zero_body(r, _):
        deg_v[pl.ds(r * 16, 16)] = zeros16
        return 0

    lax.fori_loop(0, NPAD // 16, zero_body, 0)
    pltpu.sync_copy(dst_hbm.at[wid], dst_v)

    def step(j, _):
        for i in range(KE // 16):
            idx = dst_v[j, pl.ds(i * 16, 16)]
            plsc.addupdate_scatter(deg_v, [idx], ones16)
        return 0

    lax.fori_loop(0, NSTEP, step, 0)
    pltpu.sync_copy(deg_v, out_hbm.at[pl.ds(wid * NPAD, NPAD)])


def _edge_body(q_hbm, src_hbm, dst_hbm, out_hbm,
               src_v, dst_v, rows, gsems, ssems, zeros_v, acc_sh, q_sh):
    c = lax.axis_index("c")
    s = lax.axis_index("s")
    wid = c * NS + s
    zeros16 = jnp.zeros((16,), jnp.float32)

    def zero_body(r, _):
        for i in range(H // 16):
            zeros_v[r, pl.ds(i * 16, 16)] = zeros16
        return 0

    lax.fori_loop(0, RZ, zero_body, 0)
    for t in range(RPT // RZ):
        pltpu.sync_copy(zeros_v, acc_sh.at[pl.ds(s * RPT + t * RZ, RZ)])

    @pl.when(s < NBLK)
    def _():
        sl = pl.ds(s * BN, BN)
        pltpu.sync_copy(q_hbm.at[sl], q_sh.at[sl])

    plsc.subcore_barrier()

    pltpu.sync_copy(src_hbm.at[wid], src_v)
    pltpu.sync_copy(dst_hbm.at[wid], dst_v)

    def fire_gather(b, j):
        pltpu.async_copy(q_sh.at[src_v.at[j]], rows.at[b], gsems[b])

    def wait_gather(b):
        pltpu.make_async_copy(
            q_sh.at[pl.ds(0, KE)], rows.at[b], gsems[b]).wait()

    fire_gather(0, 0)

    def pair(j2, _):
        j = 2 * j2
        fire_gather(1, j + 1)
        wait_gather(0)
        pltpu.sync_copy(rows.at[0], acc_sh.at[dst_v.at[j]], add=True)
        fire_gather(0, j + 2)
        wait_gather(1)
        pltpu.sync_copy(rows.at[1], acc_sh.at[dst_v.at[j + 1]], add=True)
        return 0

    lax.fori_loop(0, (NSTEP - 1) // 2, pair, 0)
    wait_gather(0)
    pltpu.sync_copy(rows.at[0], acc_sh.at[dst_v.at[NSTEP - 1]], add=True)

    plsc.subcore_barrier()
    for t in range(RPT // RZ):
        sl = pl.ds(s * RPT + t * RZ, RZ)
        pltpu.sync_copy(acc_sh.at[sl], out_hbm.at[c, sl])


@functools.lru_cache(maxsize=None)
def _sc_kernels():
    mesh = plsc.VectorSubcoreMesh(
        core_axis_name="c", subcore_axis_name="s",
        num_cores=NC, num_subcores=NS)
    params = pltpu.CompilerParams(needs_layout_passes=False,
                                  use_tc_tiling_on_sc=False)
    deg = pl.kernel(
        _deg_body,
        out_type=jax.ShapeDtypeStruct((NW * NPAD,), jnp.float32),
        mesh=mesh,
        compiler_params=params,
        scratch_types=[
            pltpu.VMEM((NSTEP, KE), jnp.int32),
            pltpu.VMEM((NPAD,), jnp.float32),
        ],
    )
    edge = pl.kernel(
        _edge_body,
        out_type=jax.ShapeDtypeStruct((NC, NPAD, H), jnp.float32),
        mesh=mesh,
        compiler_params=params,
        scratch_types=[
            pltpu.VMEM((NSTEP, KE), jnp.int32),
            pltpu.VMEM((NSTEP, KE), jnp.int32),
            pltpu.VMEM((2, KE, H), jnp.float32),
            [pltpu.SemaphoreType.DMA] * 2,
            [pltpu.SemaphoreType.DMA] * 2,
            pltpu.VMEM((RZ, H), jnp.float32),
            pltpu.VMEM_SHARED((NPAD, H), jnp.float32),
            pltpu.VMEM_SHARED((N, H), jnp.float32),
        ],
    )
    return deg, edge


def _prescale_body(x_ref, w_ref, degt_ref, q_ref, dinv_ref):
    deg = jnp.sum(degt_ref[...], axis=1, keepdims=True) + 1.0
    dinv = lax.rsqrt(deg)
    q_ref[...] = jnp.dot(x_ref[...], w_ref[...],
                         preferred_element_type=jnp.float32) * dinv
    dinv_ref[...] = dinv


_prescale = pl.pallas_call(
    _prescale_body,
    grid=(NBLK,),
    in_specs=[
        pl.BlockSpec((BN, D), lambda i: (i, 0)),
        pl.BlockSpec((D, H), lambda i: (0, 0)),
        pl.BlockSpec((BN, NW), lambda i: (i, 0)),
    ],
    out_specs=[
        pl.BlockSpec((BN, H), lambda i: (i, 0)),
        pl.BlockSpec((BN, 1), lambda i: (i, 0)),
    ],
    out_shape=[
        jax.ShapeDtypeStruct((N, H), jnp.float32),
        jax.ShapeDtypeStruct((N, 1), jnp.float32),
    ],
)


def _mid_body(acc_ref, q1_ref, dinv_ref, b1_ref, w2_ref, q2_ref):
    dinv = dinv_ref[...]
    a = acc_ref[0] + acc_ref[1] + q1_ref[...]
    h1 = jnp.maximum(a * dinv + b1_ref[...], 0.0)
    q2_ref[...] = jnp.dot(h1, w2_ref[...],
                          preferred_element_type=jnp.float32) * dinv


_mid = pl.pallas_call(
    _mid_body,
    grid=(NBLK,),
    in_specs=[
        pl.BlockSpec((NC, BN, H), lambda i: (0, i, 0)),
        pl.BlockSpec((BN, H), lambda i: (i, 0)),
        pl.BlockSpec((BN, 1), lambda i: (i, 0)),
        pl.BlockSpec((1, H), lambda i: (0, 0)),
        pl.BlockSpec((H, H), lambda i: (0, 0)),
    ],
    out_specs=pl.BlockSpec((BN, H), lambda i: (i, 0)),
    out_shape=jax.ShapeDtypeStruct((N, H), jnp.float32),
)


def _final_body(acc_ref, q2_ref, dinv_ref, b2_ref, batch_ref,
                wc1_ref, bc1_ref, wc2_ref, bc2_ref, out_ref,
                sums_scr, cnt_scr):
    i = pl.program_id(0)

    @pl.when(i == 0)
    def _():
        sums_scr[...] = jnp.zeros((G, H), jnp.float32)
        cnt_scr[...] = jnp.zeros((G, H), jnp.float32)

    a = acc_ref[0] + acc_ref[1] + q2_ref[...]
    h2 = jnp.maximum(a * dinv_ref[...] + b2_ref[...], 0.0)
    gids = lax.broadcasted_iota(jnp.int32, (1, G), 1)
    onehot = (batch_ref[...] == gids).astype(jnp.float32)
    dn = (((0,), (0,)), ((), ()))
    sums_scr[...] += lax.dot_general(onehot, h2, dn,
                                     preferred_element_type=jnp.float32)
    cnt_scr[...] += lax.dot_general(onehot, jnp.ones_like(h2), dn,
                                    preferred_element_type=jnp.float32)

    @pl.when(i == NBLK - 1)
    def _():
        pooled = sums_scr[...] / jnp.maximum(cnt_scr[...], 1.0)
        z = jnp.maximum(jnp.dot(pooled, wc1_ref[...],
                                preferred_element_type=jnp.float32)
                        + bc1_ref[...], 0.0)
        logit = jnp.dot(z, wc2_ref[...],
                        preferred_element_type=jnp.float32) + bc2_ref[...]
        out_ref[...] = 1.0 / (1.0 + jnp.exp(-logit))


_final = pl.pallas_call(
    _final_body,
    grid=(NBLK,),
    in_specs=[
        pl.BlockSpec((NC, BN, H), lambda i: (0, i, 0)),
        pl.BlockSpec((BN, H), lambda i: (i, 0)),
        pl.BlockSpec((BN, 1), lambda i: (i, 0)),
        pl.BlockSpec((1, H), lambda i: (0, 0)),
        pl.BlockSpec((BN, 1), lambda i: (i, 0)),
        pl.BlockSpec((H, H), lambda i: (0, 0)),
        pl.BlockSpec((1, H), lambda i: (0, 0)),
        pl.BlockSpec((H, 1), lambda i: (0, 0)),
        pl.BlockSpec((1, 1), lambda i: (0, 0)),
    ],
    out_specs=pl.BlockSpec((G, 1), lambda i: (0, 0)),
    out_shape=jax.ShapeDtypeStruct((G, 1), jnp.float32),
    scratch_shapes=[
        pltpu.VMEM((G, H), jnp.float32),
        pltpu.VMEM((G, H), jnp.float32),
    ],
    compiler_params=pltpu.CompilerParams(
        dimension_semantics=("arbitrary",)),
)


def kernel(x, edge_index, batch, W1, b1, W2, b2, Wc1, bc1, Wc2, bc2):
    deg_k, edge_k = _sc_kernels()
    src3 = edge_index[0].reshape(NW, NSTEP, KE)
    dst3 = edge_index[1].reshape(NW, NSTEP, KE)
    degp = deg_k(dst3)
    degt = degp.reshape(NW, NPAD).T
    q1, dinv = _prescale(x, W1, degt)
    acc1 = edge_k(q1, src3, dst3)
    q2 = _mid(acc1, q1, dinv, b1.reshape(1, H), W2)
    acc2 = edge_k(q2, src3, dst3)
    out = _final(acc2, q2, dinv, b2.reshape(1, H), batch.reshape(N, 1),
                 Wc1, bc1.reshape(1, H), Wc2, bc2.reshape(1, 1))
    return out


# speedup vs baseline: 2.2153x; 1.1212x over previous
import functools

import jax
import jax.numpy as jnp
from jax import lax
from jax.experimental import pallas as pl
from jax.experimental.pallas import tpu as pltpu
from jax.experimental.pallas import tpu_sc as plsc

N = 10000
E = 320000
D = 128
H = 64
G = 64
NC = 2
NS = 16
NW = NC * NS
EPT = E // NW
KE = 80
NSTEP = EPT // KE
NPAD = 10240
RPT = NPAD // NS
RZ = 128
NBLK = 10
BN = N // NBLK


def _deg_body(dst_hbm, out_hbm, dst_v, deg_v):
    c = lax.axis_index("c")
    s = lax.axis_index("s")
    wid = c * NS + s
    zeros16 = jnp.zeros((16,), jnp.float32)
    ones16 = jnp.ones((16,), jnp.float32)

    def zero_body(r, _):
        deg_v[pl.ds(r * 16, 16)] = zeros16
        return 0

    lax.fori_loop(0, NPAD // 16, zero_body, 0)
    pltpu.sync_copy(dst_hbm.at[wid], dst_v)

    def step(j, _):
        for i in range(KE // 16):
            idx = dst_v[j, pl.ds(i * 16, 16)]
            plsc.addupdate_scatter(deg_v, [idx], ones16)
        return 0

    lax.fori_loop(0, NSTEP, step, 0)
    pltpu.sync_copy(deg_v, out_hbm.at[pl.ds(wid * NPAD, NPAD)])


def _edge_body(q_hbm, src_hbm, dst_hbm, out_hbm,
               src_v, dst_v, rows, gsems, ssems, zeros_v, acc_sh, q_sh):
    c = lax.axis_index("c")
    s = lax.axis_index("s")
    wid = c * NS + s
    zeros16 = jnp.zeros((16,), jnp.float32)

    def zero_body(r, _):
        for i in range(H // 16):
            zeros_v[r, pl.ds(i * 16, 16)] = zeros16
        return 0

    lax.fori_loop(0, RZ, zero_body, 0)
    for t in range(RPT // RZ):
        pltpu.sync_copy(zeros_v, acc_sh.at[pl.ds(s * RPT + t * RZ, RZ)])

    @pl.when(s < NBLK)
    def _():
        sl = pl.ds(s * BN, BN)
        pltpu.sync_copy(q_hbm.at[sl], q_sh.at[sl])

    plsc.subcore_barrier()

    pltpu.sync_copy(src_hbm.at[wid], src_v)
    pltpu.sync_copy(dst_hbm.at[wid], dst_v)

    def fire_gather(b, j):
        pltpu.async_copy(q_sh.at[src_v.at[j]], rows.at[b], gsems[b])

    def wait_gather(b):
        pltpu.make_async_copy(
            q_sh.at[pl.ds(0, KE)], rows.at[b], gsems[b]).wait()

    def fire_scatter(b, j):
        pltpu.async_copy(rows.at[b], acc_sh.at[dst_v.at[j]], ssems[b],
                         add=True)

    def wait_scatter(b):
        pltpu.make_async_copy(
            rows.at[b], acc_sh.at[pl.ds(0, KE)], ssems[b]).wait()

    def step(j, b, do_swait, fire_next):
        wait_gather(b)
        fire_scatter(b, j)
        if do_swait:
            wait_scatter((b + 2) % 4)
        if fire_next:
            fire_gather((b + 2) % 4, j + 2)

    fire_gather(0, 0)
    fire_gather(1, 1)
    step(0, 0, False, True)
    step(1, 1, False, True)
    step(2, 2, True, True)
    step(3, 3, True, True)
    step(4, 0, True, True)
    step(5, 1, True, True)

    def block(i, _):
        j0 = 6 + 4 * i
        for k in range(4):
            step(j0 + k, (6 + k) % 4, True, True)
        return 0

    lax.fori_loop(0, 29, block, 0)
    step(NSTEP - 3, (NSTEP - 3) % 4, True, True)
    step(NSTEP - 2, (NSTEP - 2) % 4, True, False)
    step(NSTEP - 1, (NSTEP - 1) % 4, True, False)
    wait_scatter((NSTEP - 2) % 4)
    wait_scatter((NSTEP - 1) % 4)

    plsc.subcore_barrier()
    for t in range(RPT // RZ):
        sl = pl.ds(s * RPT + t * RZ, RZ)
        pltpu.sync_copy(acc_sh.at[sl], out_hbm.at[c, sl])


@functools.lru_cache(maxsize=None)
def _sc_kernels():
    mesh = plsc.VectorSubcoreMesh(
        core_axis_name="c", subcore_axis_name="s",
        num_cores=NC, num_subcores=NS)
    params = pltpu.CompilerParams(needs_layout_passes=False,
                                  use_tc_tiling_on_sc=False)
    deg = pl.kernel(
        _deg_body,
        out_type=jax.ShapeDtypeStruct((NW * NPAD,), jnp.float32),
        mesh=mesh,
        compiler_params=params,
        scratch_types=[
            pltpu.VMEM((NSTEP, KE), jnp.int32),
            pltpu.VMEM((NPAD,), jnp.float32),
        ],
    )
    edge = pl.kernel(
        _edge_body,
        out_type=jax.ShapeDtypeStruct((NC, NPAD, H), jnp.float32),
        mesh=mesh,
        compiler_params=params,
        scratch_types=[
            pltpu.VMEM((NSTEP, KE), jnp.int32),
            pltpu.VMEM((NSTEP, KE), jnp.int32),
            pltpu.VMEM((4, KE, H), jnp.float32),
            [pltpu.SemaphoreType.DMA] * 4,
            [pltpu.SemaphoreType.DMA] * 4,
            pltpu.VMEM((RZ, H), jnp.float32),
            pltpu.VMEM_SHARED((NPAD, H), jnp.float32),
            pltpu.VMEM_SHARED((N, H), jnp.float32),
        ],
    )
    return deg, edge


def _prescale_body(x_ref, w_ref, degt_ref, q_ref, dinv_ref):
    deg = jnp.sum(degt_ref[...], axis=1, keepdims=True) + 1.0
    dinv = lax.rsqrt(deg)
    q_ref[...] = jnp.dot(x_ref[...], w_ref[...],
                         preferred_element_type=jnp.float32) * dinv
    dinv_ref[...] = dinv


_prescale = pl.pallas_call(
    _prescale_body,
    grid=(NBLK,),
    in_specs=[
        pl.BlockSpec((BN, D), lambda i: (i, 0)),
        pl.BlockSpec((D, H), lambda i: (0, 0)),
        pl.BlockSpec((BN, NW), lambda i: (i, 0)),
    ],
    out_specs=[
        pl.BlockSpec((BN, H), lambda i: (i, 0)),
        pl.BlockSpec((BN, 1), lambda i: (i, 0)),
    ],
    out_shape=[
        jax.ShapeDtypeStruct((N, H), jnp.float32),
        jax.ShapeDtypeStruct((N, 1), jnp.float32),
    ],
)


def _mid_body(acc_ref, q1_ref, dinv_ref, b1_ref, w2_ref, q2_ref):
    dinv = dinv_ref[...]
    a = acc_ref[0] + acc_ref[1] + q1_ref[...]
    h1 = jnp.maximum(a * dinv + b1_ref[...], 0.0)
    q2_ref[...] = jnp.dot(h1, w2_ref[...],
                          preferred_element_type=jnp.float32) * dinv


_mid = pl.pallas_call(
    _mid_body,
    grid=(NBLK,),
    in_specs=[
        pl.BlockSpec((NC, BN, H), lambda i: (0, i, 0)),
        pl.BlockSpec((BN, H), lambda i: (i, 0)),
        pl.BlockSpec((BN, 1), lambda i: (i, 0)),
        pl.BlockSpec((1, H), lambda i: (0, 0)),
        pl.BlockSpec((H, H), lambda i: (0, 0)),
    ],
    out_specs=pl.BlockSpec((BN, H), lambda i: (i, 0)),
    out_shape=jax.ShapeDtypeStruct((N, H), jnp.float32),
)


def _final_body(acc_ref, q2_ref, dinv_ref, b2_ref, batch_ref,
                wc1_ref, bc1_ref, wc2_ref, bc2_ref, out_ref,
                sums_scr, cnt_scr):
    i = pl.program_id(0)

    @pl.when(i == 0)
    def _():
        sums_scr[...] = jnp.zeros((G, H), jnp.float32)
        cnt_scr[...] = jnp.zeros((G, H), jnp.float32)

    a = acc_ref[0] + acc_ref[1] + q2_ref[...]
    h2 = jnp.maximum(a * dinv_ref[...] + b2_ref[...], 0.0)
    gids = lax.broadcasted_iota(jnp.int32, (1, G), 1)
    onehot = (batch_ref[...] == gids).astype(jnp.float32)
    dn = (((0,), (0,)), ((), ()))
    sums_scr[...] += lax.dot_general(onehot, h2, dn,
                                     preferred_element_type=jnp.float32)
    cnt_scr[...] += lax.dot_general(onehot, jnp.ones_like(h2), dn,
                                    preferred_element_type=jnp.float32)

    @pl.when(i == NBLK - 1)
    def _():
        pooled = sums_scr[...] / jnp.maximum(cnt_scr[...], 1.0)
        z = jnp.maximum(jnp.dot(pooled, wc1_ref[...],
                                preferred_element_type=jnp.float32)
                        + bc1_ref[...], 0.0)
        logit = jnp.dot(z, wc2_ref[...],
                        preferred_element_type=jnp.float32) + bc2_ref[...]
        out_ref[...] = 1.0 / (1.0 + jnp.exp(-logit))


_final = pl.pallas_call(
    _final_body,
    grid=(NBLK,),
    in_specs=[
        pl.BlockSpec((NC, BN, H), lambda i: (0, i, 0)),
        pl.BlockSpec((BN, H), lambda i: (i, 0)),
        pl.BlockSpec((BN, 1), lambda i: (i, 0)),
        pl.BlockSpec((1, H), lambda i: (0, 0)),
        pl.BlockSpec((BN, 1), lambda i: (i, 0)),
        pl.BlockSpec((H, H), lambda i: (0, 0)),
        pl.BlockSpec((1, H), lambda i: (0, 0)),
        pl.BlockSpec((H, 1), lambda i: (0, 0)),
        pl.BlockSpec((1, 1), lambda i: (0, 0)),
    ],
    out_specs=pl.BlockSpec((G, 1), lambda i: (0, 0)),
    out_shape=jax.ShapeDtypeStruct((G, 1), jnp.float32),
    scratch_shapes=[
        pltpu.VMEM((G, H), jnp.float32),
        pltpu.VMEM((G, H), jnp.float32),
    ],
    compiler_params=pltpu.CompilerParams(
        dimension_semantics=("arbitrary",)),
)


def kernel(x, edge_index, batch, W1, b1, W2, b2, Wc1, bc1, Wc2, bc2):
    deg_k, edge_k = _sc_kernels()
    src3 = edge_index[0].reshape(NW, NSTEP, KE)
    dst3 = edge_index[1].reshape(NW, NSTEP, KE)
    degp = deg_k(dst3)
    degt = degp.reshape(NW, NPAD).T
    q1, dinv = _prescale(x, W1, degt)
    acc1 = edge_k(q1, src3, dst3)
    q2 = _mid(acc1, q1, dinv, b1.reshape(1, H), W2)
    acc2 = edge_k(q2, src3, dst3)
    out = _final(acc2, q2, dinv, b2.reshape(1, H), batch.reshape(N, 1),
                 Wc1, bc1.reshape(1, H), Wc2, bc2.reshape(1, 1))
    return out


# speedup vs baseline: 2.3352x; 1.0541x over previous
import functools

import jax
import jax.numpy as jnp
from jax import lax
from jax.experimental import pallas as pl
from jax.experimental.pallas import tpu as pltpu
from jax.experimental.pallas import tpu_sc as plsc

N = 10000
E = 320000
D = 128
H = 64
G = 64
NC = 2
NS = 16
NW = NC * NS
EPT = E // NW
KE = 80
NSTEP = EPT // KE
NBUF = 8
DEPTH = NBUF // 2
NPAD = 10240
RPT = NPAD // NS
RZ = 128
NBLK = 10
BN = N // NBLK


def _deg_body(dst_hbm, out_hbm, dst_v, deg_v):
    c = lax.axis_index("c")
    s = lax.axis_index("s")
    wid = c * NS + s
    zeros16 = jnp.zeros((16,), jnp.float32)
    ones16 = jnp.ones((16,), jnp.float32)

    def zero_body(r, _):
        deg_v[pl.ds(r * 16, 16)] = zeros16
        return 0

    lax.fori_loop(0, NPAD // 16, zero_body, 0)
    pltpu.sync_copy(dst_hbm.at[wid], dst_v)

    def step(j, _):
        for i in range(KE // 16):
            idx = dst_v[j, pl.ds(i * 16, 16)]
            plsc.addupdate_scatter(deg_v, [idx], ones16)
        return 0

    lax.fori_loop(0, NSTEP, step, 0)
    pltpu.sync_copy(deg_v, out_hbm.at[pl.ds(wid * NPAD, NPAD)])


def _edge_body(q_hbm, src_hbm, dst_hbm, out_hbm,
               src_v, dst_v, rows, gsems, ssems, zeros_v, acc_sh):
    c = lax.axis_index("c")
    s = lax.axis_index("s")
    wid = c * NS + s
    zeros16 = jnp.zeros((16,), jnp.float32)

    def zero_body(r, _):
        for i in range(H // 16):
            zeros_v[r, pl.ds(i * 16, 16)] = zeros16
        return 0

    lax.fori_loop(0, RZ, zero_body, 0)
    for t in range(RPT // RZ):
        pltpu.sync_copy(zeros_v, acc_sh.at[pl.ds(s * RPT + t * RZ, RZ)])
    plsc.subcore_barrier()

    pltpu.sync_copy(src_hbm.at[wid], src_v)
    pltpu.sync_copy(dst_hbm.at[wid], dst_v)

    def fire_gather(b, j):
        pltpu.async_copy(q_hbm.at[src_v.at[j]], rows.at[b], gsems[b])

    def wait_gather(b):
        pltpu.make_async_copy(
            q_hbm.at[pl.ds(0, KE)], rows.at[b], gsems[b]).wait()

    def fire_scatter(b, j):
        pltpu.async_copy(rows.at[b], acc_sh.at[dst_v.at[j]], ssems[b],
                         add=True)

    def wait_scatter(b):
        pltpu.make_async_copy(
            rows.at[b], acc_sh.at[pl.ds(0, KE)], ssems[b]).wait()

    def step(j, b, do_swait, fire_next):
        wait_gather(b)
        fire_scatter(b, j)
        if do_swait:
            wait_scatter((b + DEPTH) % NBUF)
        if fire_next:
            fire_gather((b + DEPTH) % NBUF, j + DEPTH)

    for j in range(DEPTH):
        fire_gather(j, j)
    for j in range(DEPTH):
        step(j, j, False, True)

    def block(i, _):
        j0 = DEPTH + NBUF * i
        for k in range(NBUF):
            step(j0 + k, (DEPTH + k) % NBUF, True, True)
        return 0

    nblk_sc = (NSTEP - 2 * DEPTH - 1) // NBUF
    lax.fori_loop(0, nblk_sc, block, 0)
    for j in range(DEPTH + NBUF * nblk_sc, NSTEP):
        step(j, j % NBUF, True, j + DEPTH < NSTEP)
    for j in range(NSTEP - DEPTH, NSTEP):
        wait_scatter(j % NBUF)

    plsc.subcore_barrier()
    for t in range(RPT // RZ):
        sl = pl.ds(s * RPT + t * RZ, RZ)
        pltpu.sync_copy(acc_sh.at[sl], out_hbm.at[c, sl])


@functools.lru_cache(maxsize=None)
def _sc_kernels():
    mesh = plsc.VectorSubcoreMesh(
        core_axis_name="c", subcore_axis_name="s",
        num_cores=NC, num_subcores=NS)
    params = pltpu.CompilerParams(needs_layout_passes=False,
                                  use_tc_tiling_on_sc=False)
    deg = pl.kernel(
        _deg_body,
        out_type=jax.ShapeDtypeStruct((NW * NPAD,), jnp.float32),
        mesh=mesh,
        compiler_params=params,
        scratch_types=[
            pltpu.VMEM((NSTEP, KE), jnp.int32),
            pltpu.VMEM((NPAD,), jnp.float32),
        ],
    )
    edge = pl.kernel(
        _edge_body,
        out_type=jax.ShapeDtypeStruct((NC, NPAD, H), jnp.float32),
        mesh=mesh,
        compiler_params=params,
        scratch_types=[
            pltpu.VMEM((NSTEP, KE), jnp.int32),
            pltpu.VMEM((NSTEP, KE), jnp.int32),
            pltpu.VMEM((NBUF, KE, H), jnp.float32),
            [pltpu.SemaphoreType.DMA] * NBUF,
            [pltpu.SemaphoreType.DMA] * NBUF,
            pltpu.VMEM((RZ, H), jnp.float32),
            pltpu.VMEM_SHARED((NPAD, H), jnp.float32),
        ],
    )
    return deg, edge


def _prescale_body(x_ref, w_ref, degt_ref, q_ref, dinv_ref):
    deg = jnp.sum(degt_ref[...], axis=1, keepdims=True) + 1.0
    dinv = lax.rsqrt(deg)
    q_ref[...] = jnp.dot(x_ref[...], w_ref[...],
                         preferred_element_type=jnp.float32) * dinv
    dinv_ref[...] = dinv


_prescale = pl.pallas_call(
    _prescale_body,
    grid=(NBLK,),
    in_specs=[
        pl.BlockSpec((BN, D), lambda i: (i, 0)),
        pl.BlockSpec((D, H), lambda i: (0, 0)),
        pl.BlockSpec((BN, NW), lambda i: (i, 0)),
    ],
    out_specs=[
        pl.BlockSpec((BN, H), lambda i: (i, 0)),
        pl.BlockSpec((BN, 1), lambda i: (i, 0)),
    ],
    out_shape=[
        jax.ShapeDtypeStruct((N, H), jnp.float32),
        jax.ShapeDtypeStruct((N, 1), jnp.float32),
    ],
)


def _mid_body(acc_ref, q1_ref, dinv_ref, b1_ref, w2_ref, q2_ref):
    dinv = dinv_ref[...]
    a = acc_ref[0] + acc_ref[1] + q1_ref[...]
    h1 = jnp.maximum(a * dinv + b1_ref[...], 0.0)
    q2_ref[...] = jnp.dot(h1, w2_ref[...],
                          preferred_element_type=jnp.float32) * dinv


_mid = pl.pallas_call(
    _mid_body,
    grid=(NBLK,),
    in_specs=[
        pl.BlockSpec((NC, BN, H), lambda i: (0, i, 0)),
        pl.BlockSpec((BN, H), lambda i: (i, 0)),
        pl.BlockSpec((BN, 1), lambda i: (i, 0)),
        pl.BlockSpec((1, H), lambda i: (0, 0)),
        pl.BlockSpec((H, H), lambda i: (0, 0)),
    ],
    out_specs=pl.BlockSpec((BN, H), lambda i: (i, 0)),
    out_shape=jax.ShapeDtypeStruct((N, H), jnp.float32),
)


def _final_body(acc_ref, q2_ref, dinv_ref, b2_ref, batch_ref,
                wc1_ref, bc1_ref, wc2_ref, bc2_ref, out_ref,
                sums_scr, cnt_scr):
    i = pl.program_id(0)

    @pl.when(i == 0)
    def _():
        sums_scr[...] = jnp.zeros((G, H), jnp.float32)
        cnt_scr[...] = jnp.zeros((G, H), jnp.float32)

    a = acc_ref[0] + acc_ref[1] + q2_ref[...]
    h2 = jnp.maximum(a * dinv_ref[...] + b2_ref[...], 0.0)
    gids = lax.broadcasted_iota(jnp.int32, (1, G), 1)
    onehot = (batch_ref[...] == gids).astype(jnp.float32)
    dn = (((0,), (0,)), ((), ()))
    sums_scr[...] += lax.dot_general(onehot, h2, dn,
                                     preferred_element_type=jnp.float32)
    cnt_scr[...] += lax.dot_general(onehot, jnp.ones_like(h2), dn,
                                    preferred_element_type=jnp.float32)

    @pl.when(i == NBLK - 1)
    def _():
        pooled = sums_scr[...] / jnp.maximum(cnt_scr[...], 1.0)
        z = jnp.maximum(jnp.dot(pooled, wc1_ref[...],
                                preferred_element_type=jnp.float32)
                        + bc1_ref[...], 0.0)
        logit = jnp.dot(z, wc2_ref[...],
                        preferred_element_type=jnp.float32) + bc2_ref[...]
        out_ref[...] = 1.0 / (1.0 + jnp.exp(-logit))


_final = pl.pallas_call(
    _final_body,
    grid=(NBLK,),
    in_specs=[
        pl.BlockSpec((NC, BN, H), lambda i: (0, i, 0)),
        pl.BlockSpec((BN, H), lambda i: (i, 0)),
        pl.BlockSpec((BN, 1), lambda i: (i, 0)),
        pl.BlockSpec((1, H), lambda i: (0, 0)),
        pl.BlockSpec((BN, 1), lambda i: (i, 0)),
        pl.BlockSpec((H, H), lambda i: (0, 0)),
        pl.BlockSpec((1, H), lambda i: (0, 0)),
        pl.BlockSpec((H, 1), lambda i: (0, 0)),
        pl.BlockSpec((1, 1), lambda i: (0, 0)),
    ],
    out_specs=pl.BlockSpec((G, 1), lambda i: (0, 0)),
    out_shape=jax.ShapeDtypeStruct((G, 1), jnp.float32),
    scratch_shapes=[
        pltpu.VMEM((G, H), jnp.float32),
        pltpu.VMEM((G, H), jnp.float32),
    ],
    compiler_params=pltpu.CompilerParams(
        dimension_semantics=("arbitrary",)),
)


def kernel(x, edge_index, batch, W1, b1, W2, b2, Wc1, bc1, Wc2, bc2):
    deg_k, edge_k = _sc_kernels()
    src3 = edge_index[0].reshape(NW, NSTEP, KE)
    dst3 = edge_index[1].reshape(NW, NSTEP, KE)
    degp = deg_k(dst3)
    degt = degp.reshape(NW, NPAD).T
    q1, dinv = _prescale(x, W1, degt)
    acc1 = edge_k(q1, src3, dst3)
    q2 = _mid(acc1, q1, dinv, b1.reshape(1, H), W2)
    acc2 = edge_k(q2, src3, dst3)
    out = _final(acc2, q2, dinv, b2.reshape(1, H), batch.reshape(N, 1),
                 Wc1, bc1.reshape(1, H), Wc2, bc2.reshape(1, 1))
    return out


# speedup vs baseline: 2.4024x; 1.0288x over previous
import functools

import jax
import jax.numpy as jnp
from jax import lax
from jax.experimental import pallas as pl
from jax.experimental.pallas import tpu as pltpu
from jax.experimental.pallas import tpu_sc as plsc

N = 10000
E = 320000
D = 128
H = 64
G = 64
NC = 2
NS = 16
NW = NC * NS
EPT = E // NW
KE = 80
NSTEP = EPT // KE
NBUF = 12
DEPTH = NBUF // 2
NPAD = 10240
RPT = NPAD // NS
RZ = 128
NBLK = 10
BN = N // NBLK


def _deg_body(dst_hbm, out_hbm, dst_v, deg_v):
    c = lax.axis_index("c")
    s = lax.axis_index("s")
    wid = c * NS + s
    zeros16 = jnp.zeros((16,), jnp.float32)
    ones16 = jnp.ones((16,), jnp.float32)

    def zero_body(r, _):
        deg_v[pl.ds(r * 16, 16)] = zeros16
        return 0

    lax.fori_loop(0, NPAD // 16, zero_body, 0)
    pltpu.sync_copy(dst_hbm.at[wid], dst_v)

    def step(j, _):
        for i in range(KE // 16):
            idx = dst_v[j, pl.ds(i * 16, 16)]
            plsc.addupdate_scatter(deg_v, [idx], ones16)
        return 0

    lax.fori_loop(0, NSTEP, step, 0)
    pltpu.sync_copy(deg_v, out_hbm.at[pl.ds(wid * NPAD, NPAD)])


def _edge_body(q_hbm, src_hbm, dst_hbm, out_hbm,
               src_v, dst_v, rows, gsems, ssems, zeros_v, acc_sh):
    c = lax.axis_index("c")
    s = lax.axis_index("s")
    wid = c * NS + s
    zeros16 = jnp.zeros((16,), jnp.float32)

    def zero_body(r, _):
        for i in range(H // 16):
            zeros_v[r, pl.ds(i * 16, 16)] = zeros16
        return 0

    lax.fori_loop(0, RZ, zero_body, 0)
    for t in range(RPT // RZ):
        pltpu.sync_copy(zeros_v, acc_sh.at[pl.ds(s * RPT + t * RZ, RZ)])
    plsc.subcore_barrier()

    pltpu.sync_copy(src_hbm.at[wid], src_v)
    pltpu.sync_copy(dst_hbm.at[wid], dst_v)

    def fire_gather(b, j):
        pltpu.async_copy(q_hbm.at[src_v.at[j]], rows.at[b], gsems[b])

    def wait_gather(b):
        pltpu.make_async_copy(
            q_hbm.at[pl.ds(0, KE)], rows.at[b], gsems[b]).wait()

    def fire_scatter(b, j):
        pltpu.async_copy(rows.at[b], acc_sh.at[dst_v.at[j]], ssems[b],
                         add=True)

    def wait_scatter(b):
        pltpu.make_async_copy(
            rows.at[b], acc_sh.at[pl.ds(0, KE)], ssems[b]).wait()

    def step(j, b, do_swait, fire_next):
        wait_gather(b)
        fire_scatter(b, j)
        if do_swait:
            wait_scatter((b + DEPTH) % NBUF)
        if fire_next:
            fire_gather((b + DEPTH) % NBUF, j + DEPTH)

    for j in range(DEPTH):
        fire_gather(j, j)
    for j in range(DEPTH):
        step(j, j, False, True)

    def block(i, _):
        j0 = DEPTH + NBUF * i
        for k in range(NBUF):
            step(j0 + k, (DEPTH + k) % NBUF, True, True)
        return 0

    nblk_sc = (NSTEP - 2 * DEPTH - 1) // NBUF
    lax.fori_loop(0, nblk_sc, block, 0)
    for j in range(DEPTH + NBUF * nblk_sc, NSTEP):
        step(j, j % NBUF, True, j + DEPTH < NSTEP)
    for j in range(NSTEP - DEPTH, NSTEP):
        wait_scatter(j % NBUF)

    plsc.subcore_barrier()
    for t in range(RPT // RZ):
        sl = pl.ds(s * RPT + t * RZ, RZ)
        pltpu.sync_copy(acc_sh.at[sl], out_hbm.at[c, sl])


@functools.lru_cache(maxsize=None)
def _sc_kernels():
    mesh = plsc.VectorSubcoreMesh(
        core_axis_name="c", subcore_axis_name="s",
        num_cores=NC, num_subcores=NS)
    params = pltpu.CompilerParams(needs_layout_passes=False,
                                  use_tc_tiling_on_sc=False)
    deg = pl.kernel(
        _deg_body,
        out_type=jax.ShapeDtypeStruct((NW * NPAD,), jnp.float32),
        mesh=mesh,
        compiler_params=params,
        scratch_types=[
            pltpu.VMEM((NSTEP, KE), jnp.int32),
            pltpu.VMEM((NPAD,), jnp.float32),
        ],
    )
    edge = pl.kernel(
        _edge_body,
        out_type=jax.ShapeDtypeStruct((NC, NPAD, H), jnp.float32),
        mesh=mesh,
        compiler_params=params,
        scratch_types=[
            pltpu.VMEM((NSTEP, KE), jnp.int32),
            pltpu.VMEM((NSTEP, KE), jnp.int32),
            pltpu.VMEM((NBUF, KE, H), jnp.float32),
            [pltpu.SemaphoreType.DMA] * NBUF,
            [pltpu.SemaphoreType.DMA] * NBUF,
            pltpu.VMEM((RZ, H), jnp.float32),
            pltpu.VMEM_SHARED((NPAD, H), jnp.float32),
        ],
    )
    return deg, edge


def _prescale_body(x_ref, w_ref, degt_ref, q_ref, dinv_ref):
    deg = jnp.sum(degt_ref[...], axis=1, keepdims=True) + 1.0
    dinv = lax.rsqrt(deg)
    q_ref[...] = jnp.dot(x_ref[...], w_ref[...],
                         preferred_element_type=jnp.float32) * dinv
    dinv_ref[...] = dinv


_prescale = pl.pallas_call(
    _prescale_body,
    grid=(NBLK,),
    in_specs=[
        pl.BlockSpec((BN, D), lambda i: (i, 0)),
        pl.BlockSpec((D, H), lambda i: (0, 0)),
        pl.BlockSpec((BN, NW), lambda i: (i, 0)),
    ],
    out_specs=[
        pl.BlockSpec((BN, H), lambda i: (i, 0)),
        pl.BlockSpec((BN, 1), lambda i: (i, 0)),
    ],
    out_shape=[
        jax.ShapeDtypeStruct((N, H), jnp.float32),
        jax.ShapeDtypeStruct((N, 1), jnp.float32),
    ],
)


def _mid_body(acc_ref, q1_ref, dinv_ref, b1_ref, w2_ref, q2_ref):
    dinv = dinv_ref[...]
    a = acc_ref[0] + acc_ref[1] + q1_ref[...]
    h1 = jnp.maximum(a * dinv + b1_ref[...], 0.0)
    q2_ref[...] = jnp.dot(h1, w2_ref[...],
                          preferred_element_type=jnp.float32) * dinv


_mid = pl.pallas_call(
    _mid_body,
    grid=(NBLK,),
    in_specs=[
        pl.BlockSpec((NC, BN, H), lambda i: (0, i, 0)),
        pl.BlockSpec((BN, H), lambda i: (i, 0)),
        pl.BlockSpec((BN, 1), lambda i: (i, 0)),
        pl.BlockSpec((1, H), lambda i: (0, 0)),
        pl.BlockSpec((H, H), lambda i: (0, 0)),
    ],
    out_specs=pl.BlockSpec((BN, H), lambda i: (i, 0)),
    out_shape=jax.ShapeDtypeStruct((N, H), jnp.float32),
)


def _final_body(acc_ref, q2_ref, dinv_ref, b2_ref, batch_ref,
                wc1_ref, bc1_ref, wc2_ref, bc2_ref, out_ref,
                sums_scr, cnt_scr):
    i = pl.program_id(0)

    @pl.when(i == 0)
    def _():
        sums_scr[...] = jnp.zeros((G, H), jnp.float32)
        cnt_scr[...] = jnp.zeros((G, H), jnp.float32)

    a = acc_ref[0] + acc_ref[1] + q2_ref[...]
    h2 = jnp.maximum(a * dinv_ref[...] + b2_ref[...], 0.0)
    gids = lax.broadcasted_iota(jnp.int32, (1, G), 1)
    onehot = (batch_ref[...] == gids).astype(jnp.float32)
    dn = (((0,), (0,)), ((), ()))
    sums_scr[...] += lax.dot_general(onehot, h2, dn,
                                     preferred_element_type=jnp.float32)
    cnt_scr[...] += lax.dot_general(onehot, jnp.ones_like(h2), dn,
                                    preferred_element_type=jnp.float32)

    @pl.when(i == NBLK - 1)
    def _():
        pooled = sums_scr[...] / jnp.maximum(cnt_scr[...], 1.0)
        z = jnp.maximum(jnp.dot(pooled, wc1_ref[...],
                                preferred_element_type=jnp.float32)
                        + bc1_ref[...], 0.0)
        logit = jnp.dot(z, wc2_ref[...],
                        preferred_element_type=jnp.float32) + bc2_ref[...]
        out_ref[...] = 1.0 / (1.0 + jnp.exp(-logit))


_final = pl.pallas_call(
    _final_body,
    grid=(NBLK,),
    in_specs=[
        pl.BlockSpec((NC, BN, H), lambda i: (0, i, 0)),
        pl.BlockSpec((BN, H), lambda i: (i, 0)),
        pl.BlockSpec((BN, 1), lambda i: (i, 0)),
        pl.BlockSpec((1, H), lambda i: (0, 0)),
        pl.BlockSpec((BN, 1), lambda i: (i, 0)),
        pl.BlockSpec((H, H), lambda i: (0, 0)),
        pl.BlockSpec((1, H), lambda i: (0, 0)),
        pl.BlockSpec((H, 1), lambda i: (0, 0)),
        pl.BlockSpec((1, 1), lambda i: (0, 0)),
    ],
    out_specs=pl.BlockSpec((G, 1), lambda i: (0, 0)),
    out_shape=jax.ShapeDtypeStruct((G, 1), jnp.float32),
    scratch_shapes=[
        pltpu.VMEM((G, H), jnp.float32),
        pltpu.VMEM((G, H), jnp.float32),
    ],
    compiler_params=pltpu.CompilerParams(
        dimension_semantics=("arbitrary",)),
)


def kernel(x, edge_index, batch, W1, b1, W2, b2, Wc1, bc1, Wc2, bc2):
    deg_k, edge_k = _sc_kernels()
    src3 = edge_index[0].reshape(NW, NSTEP, KE)
    dst3 = edge_index[1].reshape(NW, NSTEP, KE)
    degp = deg_k(dst3)
    degt = degp.reshape(NW, NPAD).T
    q1, dinv = _prescale(x, W1, degt)
    acc1 = edge_k(q1, src3, dst3)
    q2 = _mid(acc1, q1, dinv, b1.reshape(1, H), W2)
    acc2 = edge_k(q2, src3, dst3)
    out = _final(acc2, q2, dinv, b2.reshape(1, H), batch.reshape(N, 1),
                 Wc1, bc1.reshape(1, H), Wc2, bc2.reshape(1, 1))
    return out
